# Initial kernel scaffold; baseline (speedup 1.0000x reference)
import jax
import jax.numpy as jnp
from jax.experimental import pallas as pl


def kernel(x, edge_index_r0, edge_index_r1, edge_index_r2, W1_0, b1_0, W1_1, b1_1, W1_2, b1_2, W2_0, b2_0, W2_1, b2_1, W2_2, b2_2, Wc, bc):
    raise NotImplementedError("write your pallas kernel here")


# speedup vs baseline: 1.5767x; 1.5767x over previous
import functools

import jax
import jax.numpy as jnp
from jax import lax
from jax.experimental import pallas as pl
from jax.experimental.pallas import tpu as pltpu
from jax.experimental.pallas import tpu_sc as plsc

N = 50000
NP = 51200
E = 200000
EP = 200704
CHUNKS = 98
STRIPE = 3200
TROW = 1024
W = 16
DUMMY = N

_mesh = plsc.VectorSubcoreMesh(core_axis_name="c", subcore_axis_name="s")


@functools.partial(
    pl.kernel,
    out_type=[jax.ShapeDtypeStruct((NP,), jnp.float32)] * 6,
    mesh=_mesh,
    scratch_types=[
        pltpu.VMEM_SHARED((NP,), jnp.float32),
        pltpu.VMEM((CHUNKS, 128), jnp.int32),
        pltpu.VMEM((128,), jnp.float32),
    ],
)
def _deg_kernel(ones_h, zer_h, s0, s1, s2, d0, d1, d2,
                o0, o1, o2, i0, i1, i2, acc, idxv, ones_v):
  cid = lax.axis_index("c")
  sid = lax.axis_index("s")
  pltpu.sync_copy(ones_h, ones_v)

  def task(eidx, out):
    def zero(i, carry):
      pltpu.sync_copy(zer_h, acc.at[pl.ds(sid * STRIPE + i * 128, 128)])
      return carry
    lax.fori_loop(0, STRIPE // 128, zero, 0)
    plsc.subcore_barrier()
    pltpu.sync_copy(eidx.at[sid], idxv)

    def body(j, carry):
      pltpu.sync_copy(ones_v, acc.at[idxv.at[j]], add=True)
      return carry
    lax.fori_loop(0, CHUNKS, body, 0)
    plsc.subcore_barrier()
    pltpu.sync_copy(acc.at[pl.ds(sid * STRIPE, STRIPE)],
                    out.at[pl.ds(sid * STRIPE, STRIPE)])
    plsc.subcore_barrier()

  @pl.when(cid == 0)
  def _():
    task(s0, o0)
    task(s1, o1)
    task(s2, o2)

  @pl.when(cid == 1)
  def _():
    task(d0, i0)
    task(d1, i1)
    task(d2, i2)


def _make_agg(tasks0, tasks1, nf):
  @functools.partial(
      pl.kernel,
      out_type=[jax.ShapeDtypeStruct((NP, W), jnp.float32)] * nf,
      mesh=_mesh,
      scratch_types=[
          pltpu.VMEM_SHARED((NP, W), jnp.float32),
          pltpu.VMEM((CHUNKS, 128), jnp.int32),
          pltpu.VMEM((CHUNKS, 128), jnp.int32),
          pltpu.VMEM((128, W), jnp.float32),
          pltpu.VMEM((128, W), jnp.float32),
          pltpu.SemaphoreType.DMA,
      ],
      compiler_params=pltpu.CompilerParams(use_tc_tiling_on_sc=False),
  )
  def agg_kernel(*refs):
    zer_h = refs[0]
    feats = refs[1:1 + nf]
    srcs = refs[1 + nf:4 + nf]
    dsts = refs[4 + nf:7 + nf]
    outs = refs[7 + nf:7 + 2 * nf]
    acc, idxs, idxd, rows, zvm, sem = refs[7 + 2 * nf:]
    cid = lax.axis_index("c")
    sid = lax.axis_index("s")
    pltpu.sync_copy(zer_h, zvm)

    def task(feat, src, dst, out):
      def zero(i, carry):
        pltpu.sync_copy(zvm, acc.at[pl.ds(sid * STRIPE + i * 128, 128)])
        return carry
      lax.fori_loop(0, STRIPE // 128, zero, 0)
      plsc.subcore_barrier()
      pltpu.sync_copy(src.at[sid], idxs)
      pltpu.sync_copy(dst.at[sid], idxd)

      def body(j, carry):
        pltpu.async_copy(feat.at[idxs.at[j]], rows, sem).wait()
        pltpu.sync_copy(rows, acc.at[idxd.at[j]], add=True)
        return carry
      lax.fori_loop(0, CHUNKS, body, 0)
      plsc.subcore_barrier()
      pltpu.sync_copy(acc.at[pl.ds(sid * STRIPE, STRIPE)],
                      out.at[pl.ds(sid * STRIPE, STRIPE)])
      plsc.subcore_barrier()

    @pl.when(cid == 0)
    def _():
      for f, r, o in tasks0:
        task(feats[f], srcs[r], dsts[r], outs[o])

    @pl.when(cid == 1)
    def _():
      for f, r, o in tasks1:
        task(feats[f], srcs[r], dsts[r], outs[o])

  return agg_kernel


_L1_TASKS = [(8 * r + c, r, 8 * r + c) for r in range(3) for c in range(8)]
_agg_l1 = _make_agg(_L1_TASKS[:12], _L1_TASKS[12:], 24)

_L2_TASKS = [(3 * r + c, r, 3 * r + c) for r in range(3) for c in range(3)]
_agg_l2 = _make_agg(_L2_TASKS[:5], _L2_TASKS[5:], 9)


def _scale_body(d0, d1, d2, x_ref, *outs):
  xv = x_ref[...]
  k = 0
  for dref in (d0, d1, d2):
    s = lax.rsqrt(jnp.maximum(dref[...], 1.0))
    xf = xv * s
    for c in range(8):
      outs[k][...] = xf[:, c * 16:(c + 1) * 16]
      k += 1


def _l1_body(i0, i1, i2, b10, b11, b12, w10, w11, w12, *args):
  aggs = args[:24]
  h_ref = args[24]
  acc = jnp.zeros((TROW, 128), jnp.float32)
  for r, (iref, wref) in enumerate(((i0, w10), (i1, w11), (i2, w12))):
    a = jnp.concatenate([aggs[8 * r + c][...] for c in range(8)], axis=1)
    s = lax.rsqrt(jnp.maximum(iref[...], 1.0))
    acc += jnp.dot(a * s, wref[...], preferred_element_type=jnp.float32)
  bbar = (b10[...] + b11[...] + b12[...]) * (1.0 / 3.0)
  h_ref[...] = jnp.maximum(acc * (1.0 / 3.0) + bbar, 0.0)


def _l2pre_body(o0, o1, o2, wc_ref, w20, w21, w22, h_ref, *outs):
  hv = h_ref[...]
  wc = wc_ref[...]
  wcp = jnp.concatenate([wc, jnp.zeros((128, 8), jnp.float32)], axis=1)
  for r, (oref, wref) in enumerate(((o0, w20), (o1, w21), (o2, w22))):
    m = jnp.dot(wref[...], wcp, preferred_element_type=jnp.float32)
    s = lax.rsqrt(jnp.maximum(oref[...], 1.0))
    q = jnp.dot(hv * s, m, preferred_element_type=jnp.float32)
    for c in range(3):
      outs[3 * r + c][...] = q[:, c * 16:(c + 1) * 16]


def _final_body(i0, i1, i2, b20, b21, b22, wc_ref, bc_ref, *args):
  aggs = args[:9]
  out = args[9]
  acc = jnp.zeros((TROW, 48), jnp.float32)
  for r, iref in enumerate((i0, i1, i2)):
    a = jnp.concatenate([aggs[3 * r + c][...] for c in range(3)], axis=1)
    s = lax.rsqrt(jnp.maximum(iref[...], 1.0))
    acc += a * s
  bb = jnp.dot((b20[...] + b21[...] + b22[...]) * (1.0 / 3.0), wc_ref[...],
               preferred_element_type=jnp.float32)
  out[...] = acc[:, :40] * (1.0 / 3.0) + bb + bc_ref[...]


def _row_spec(w):
  return pl.BlockSpec((TROW, w), lambda i: (i, 0))


def _full_spec(a, b):
  return pl.BlockSpec((a, b), lambda i: (0, 0))


def kernel(x, edge_index_r0, edge_index_r1, edge_index_r2,
           W1_0, b1_0, W1_1, b1_1, W1_2, b1_2,
           W2_0, b2_0, W2_1, b2_1, W2_2, b2_2,
           Wc, bc):
  xp = jnp.pad(x, ((0, NP - N), (0, 0)))

  def prep(ei):
    pad = jnp.full((EP - E,), DUMMY, jnp.int32)
    s = jnp.concatenate([ei[0].astype(jnp.int32), pad]).reshape(16, CHUNKS, 128)
    d = jnp.concatenate([ei[1].astype(jnp.int32), pad]).reshape(16, CHUNKS, 128)
    return s, d

  s0, d0 = prep(edge_index_r0)
  s1, d1 = prep(edge_index_r1)
  s2, d2 = prep(edge_index_r2)

  ones128 = jnp.ones((128,), jnp.float32)
  zer128 = jnp.zeros((128,), jnp.float32)
  zer2d = jnp.zeros((128, W), jnp.float32)

  od0, od1, od2, id0, id1, id2 = _deg_kernel(
      ones128, zer128, s0, s1, s2, d0, d1, d2)
  od = [d.reshape(NP, 1) for d in (od0, od1, od2)]
  idg = [d.reshape(NP, 1) for d in (id0, id1, id2)]

  feats = pl.pallas_call(
      _scale_body,
      grid=(NP // TROW,),
      in_specs=[_row_spec(1)] * 3 + [_row_spec(128)],
      out_specs=[_row_spec(W)] * 24,
      out_shape=[jax.ShapeDtypeStruct((NP, W), jnp.float32)] * 24,
  )(od[0], od[1], od[2], xp)

  aggs1 = _agg_l1(zer2d, *feats, s0, s1, s2, d0, d1, d2)

  b1 = [b.reshape(1, 128) for b in (b1_0, b1_1, b1_2)]
  h = pl.pallas_call(
      _l1_body,
      grid=(NP // TROW,),
      in_specs=([_row_spec(1)] * 3 + [_full_spec(1, 128)] * 3
                + [_full_spec(128, 128)] * 3 + [_row_spec(W)] * 24),
      out_specs=_row_spec(128),
      out_shape=jax.ShapeDtypeStruct((NP, 128), jnp.float32),
  )(idg[0], idg[1], idg[2], *b1, W1_0, W1_1, W1_2, *aggs1)

  qs = pl.pallas_call(
      _l2pre_body,
      grid=(NP // TROW,),
      in_specs=([_row_spec(1)] * 3 + [_full_spec(128, 40)]
                + [_full_spec(128, 128)] * 3 + [_row_spec(128)]),
      out_specs=[_row_spec(W)] * 9,
      out_shape=[jax.ShapeDtypeStruct((NP, W), jnp.float32)] * 9,
  )(od[0], od[1], od[2], Wc, W2_0, W2_1, W2_2, h)

  aggs2 = _agg_l2(zer2d, *qs, s0, s1, s2, d0, d1, d2)

  b2 = [b.reshape(1, 128) for b in (b2_0, b2_1, b2_2)]
  logits = pl.pallas_call(
      _final_body,
      grid=(NP // TROW,),
      in_specs=([_row_spec(1)] * 3 + [_full_spec(1, 128)] * 3
                + [_full_spec(128, 40)] + [_full_spec(1, 40)]
                + [_row_spec(W)] * 9),
      out_specs=_row_spec(40),
      out_shape=jax.ShapeDtypeStruct((NP, 40), jnp.float32),
  )(idg[0], idg[1], idg[2], *b2, Wc, bc.reshape(1, 40), *aggs2)

  return logits[:N]


# speedup vs baseline: 1.8439x; 1.1694x over previous
import functools

import jax
import jax.numpy as jnp
from jax import lax
from jax.experimental import pallas as pl
from jax.experimental.pallas import tpu as pltpu
from jax.experimental.pallas import tpu_sc as plsc

N = 50000
NP = 51200
E = 200000
EP = 200704
CHUNKS = 98
GROUP = 7
STRIPE = 3200
TROW = 1024
W = 16
DUMMY = N

_mesh1 = plsc.VectorSubcoreMesh(
    core_axis_name="c", subcore_axis_name="s", num_cores=1)


@functools.partial(
    pl.kernel,
    out_type=[jax.ShapeDtypeStruct((NP,), jnp.float32)] * 3,
    mesh=_mesh1,
    scratch_types=[
        pltpu.VMEM_SHARED((NP,), jnp.float32),
        pltpu.VMEM((CHUNKS, 128), jnp.int32),
        pltpu.VMEM((128,), jnp.float32),
        pltpu.VMEM((STRIPE,), jnp.float32),
        pltpu.SemaphoreType.DMA,
    ],
)
def _deg_kernel(ones_h, zer_h, e0, e1, e2, o0, o1, o2,
                acc, idxv, ones_v, zvm, sem):
  sid = lax.axis_index("s")
  pltpu.sync_copy(ones_h, ones_v)
  pltpu.sync_copy(zer_h, zvm)

  def task(eidx, out):
    pltpu.sync_copy(zvm, acc.at[pl.ds(sid * STRIPE, STRIPE)])
    plsc.subcore_barrier()
    pltpu.sync_copy(eidx.at[sid], idxv)

    def body(g, carry):
      descs = [
          pltpu.async_copy(ones_v, acc.at[idxv.at[g * GROUP + k]], sem,
                           add=True)
          for k in range(GROUP)
      ]
      for d in descs:
        d.wait()
      return carry
    lax.fori_loop(0, CHUNKS // GROUP, body, 0)
    plsc.subcore_barrier()
    pltpu.sync_copy(acc.at[pl.ds(sid * STRIPE, STRIPE)],
                    out.at[pl.ds(sid * STRIPE, STRIPE)])
    plsc.subcore_barrier()

  task(e0, o0)
  task(e1, o1)
  task(e2, o2)


def _make_agg(tasks, nf):
  @functools.partial(
      pl.kernel,
      out_type=[jax.ShapeDtypeStruct((NP, W), jnp.float32)] * len(tasks),
      mesh=_mesh1,
      scratch_types=[
          pltpu.VMEM_SHARED((NP, W), jnp.float32),
          pltpu.VMEM((CHUNKS, 128), jnp.int32),
          pltpu.VMEM((CHUNKS, 128), jnp.int32),
          pltpu.VMEM((GROUP, 128, W), jnp.float32),
          pltpu.VMEM((128, W), jnp.float32),
          pltpu.SemaphoreType.DMA,
          pltpu.SemaphoreType.DMA,
      ],
      compiler_params=pltpu.CompilerParams(use_tc_tiling_on_sc=False),
  )
  def agg_kernel(*refs):
    zer_h = refs[0]
    feats = refs[1:1 + nf]
    srcs = refs[1 + nf:4 + nf]
    dsts = refs[4 + nf:7 + nf]
    outs = refs[7 + nf:7 + nf + len(tasks)]
    acc, idxs, idxd, rows, zvm, gsem, ssem = refs[7 + nf + len(tasks):]
    sid = lax.axis_index("s")
    pltpu.sync_copy(zer_h, zvm)

    def task(feat, src, dst, out):
      zd = [
          pltpu.async_copy(zvm, acc.at[pl.ds(sid * STRIPE + i * 128, 128)],
                           ssem)
          for i in range(STRIPE // 128)
      ]
      for d in zd:
        d.wait()
      plsc.subcore_barrier()
      pltpu.sync_copy(src.at[sid], idxs)
      pltpu.sync_copy(dst.at[sid], idxd)

      def body(g, carry):
        gd = [
            pltpu.async_copy(feat.at[idxs.at[g * GROUP + k]], rows.at[k],
                             gsem)
            for k in range(GROUP)
        ]
        for d in gd:
          d.wait()
        sd = [
            pltpu.async_copy(rows.at[k], acc.at[idxd.at[g * GROUP + k]],
                             ssem, add=True)
            for k in range(GROUP)
        ]
        for d in sd:
          d.wait()
        return carry
      lax.fori_loop(0, CHUNKS // GROUP, body, 0)
      plsc.subcore_barrier()
      pltpu.sync_copy(acc.at[pl.ds(sid * STRIPE, STRIPE)],
                      out.at[pl.ds(sid * STRIPE, STRIPE)])
      plsc.subcore_barrier()

    for f, r, o in tasks:
      task(feats[f], srcs[r], dsts[r], outs[o])

  return agg_kernel


_L1A = [(8 * r + c, r, 8 * r + c) for r in range(3) for c in range(8)][:12]
_L1B = [(f, r, o - 12) for (f, r, o) in
        [(8 * r + c, r, 8 * r + c) for r in range(3) for c in range(8)][12:]]
_agg_l1a = _make_agg(_L1A, 24)
_agg_l1b = _make_agg(_L1B, 24)

_L2 = [(3 * r + c, r, 3 * r + c) for r in range(3) for c in range(3)]
_L2A = _L2[:5]
_L2B = [(f, r, o - 5) for (f, r, o) in _L2[5:]]
_agg_l2a = _make_agg(_L2A, 9)
_agg_l2b = _make_agg(_L2B, 9)


def _scale_body(d0, d1, d2, x_ref, *outs):
  xv = x_ref[...]
  k = 0
  for dref in (d0, d1, d2):
    s = lax.rsqrt(jnp.maximum(dref[...], 1.0))
    xf = xv * s
    for c in range(8):
      outs[k][...] = xf[:, c * 16:(c + 1) * 16]
      k += 1


def _l1_body(i0, i1, i2, b10, b11, b12, w10, w11, w12, *args):
  aggs = args[:24]
  h_ref = args[24]
  acc = jnp.zeros((TROW, 128), jnp.float32)
  for r, (iref, wref) in enumerate(((i0, w10), (i1, w11), (i2, w12))):
    a = jnp.concatenate([aggs[8 * r + c][...] for c in range(8)], axis=1)
    s = lax.rsqrt(jnp.maximum(iref[...], 1.0))
    acc += jnp.dot(a * s, wref[...], preferred_element_type=jnp.float32)
  bbar = (b10[...] + b11[...] + b12[...]) * (1.0 / 3.0)
  h_ref[...] = jnp.maximum(acc * (1.0 / 3.0) + bbar, 0.0)


def _l2pre_body(o0, o1, o2, wc_ref, w20, w21, w22, h_ref, *outs):
  hv = h_ref[...]
  wc = wc_ref[...]
  wcp = jnp.concatenate([wc, jnp.zeros((128, 8), jnp.float32)], axis=1)
  for r, (oref, wref) in enumerate(((o0, w20), (o1, w21), (o2, w22))):
    m = jnp.dot(wref[...], wcp, preferred_element_type=jnp.float32)
    s = lax.rsqrt(jnp.maximum(oref[...], 1.0))
    q = jnp.dot(hv * s, m, preferred_element_type=jnp.float32)
    for c in range(3):
      outs[3 * r + c][...] = q[:, c * 16:(c + 1) * 16]


def _final_body(i0, i1, i2, b20, b21, b22, wc_ref, bc_ref, *args):
  aggs = args[:9]
  out = args[9]
  acc = jnp.zeros((TROW, 48), jnp.float32)
  for r, iref in enumerate((i0, i1, i2)):
    a = jnp.concatenate([aggs[3 * r + c][...] for c in range(3)], axis=1)
    s = lax.rsqrt(jnp.maximum(iref[...], 1.0))
    acc += a * s
  bb = jnp.dot((b20[...] + b21[...] + b22[...]) * (1.0 / 3.0), wc_ref[...],
               preferred_element_type=jnp.float32)
  out[...] = acc[:, :40] * (1.0 / 3.0) + bb + bc_ref[...]


def _row_spec(w):
  return pl.BlockSpec((TROW, w), lambda i: (i, 0))


def _full_spec(a, b):
  return pl.BlockSpec((a, b), lambda i: (0, 0))


def kernel(x, edge_index_r0, edge_index_r1, edge_index_r2,
           W1_0, b1_0, W1_1, b1_1, W1_2, b1_2,
           W2_0, b2_0, W2_1, b2_1, W2_2, b2_2,
           Wc, bc):
  xp = jnp.pad(x, ((0, NP - N), (0, 0)))

  def prep(ei):
    pad = jnp.full((EP - E,), DUMMY, jnp.int32)
    s = jnp.concatenate([ei[0].astype(jnp.int32), pad]).reshape(16, CHUNKS, 128)
    d = jnp.concatenate([ei[1].astype(jnp.int32), pad]).reshape(16, CHUNKS, 128)
    return s, d

  s0, d0 = prep(edge_index_r0)
  s1, d1 = prep(edge_index_r1)
  s2, d2 = prep(edge_index_r2)

  ones128 = jnp.ones((128,), jnp.float32)
  zer1d = jnp.zeros((STRIPE,), jnp.float32)
  zer2d = jnp.zeros((128, W), jnp.float32)

  od0, od1, od2 = _deg_kernel(ones128, zer1d, s0, s1, s2)
  id0, id1, id2 = _deg_kernel(ones128, zer1d, d0, d1, d2)
  od = [d.reshape(NP, 1) for d in (od0, od1, od2)]
  idg = [d.reshape(NP, 1) for d in (id0, id1, id2)]

  feats = pl.pallas_call(
      _scale_body,
      grid=(NP // TROW,),
      in_specs=[_row_spec(1)] * 3 + [_row_spec(128)],
      out_specs=[_row_spec(W)] * 24,
      out_shape=[jax.ShapeDtypeStruct((NP, W), jnp.float32)] * 24,
  )(od[0], od[1], od[2], xp)

  aggs1 = (list(_agg_l1a(zer2d, *feats, s0, s1, s2, d0, d1, d2))
           + list(_agg_l1b(zer2d, *feats, s0, s1, s2, d0, d1, d2)))

  b1 = [b.reshape(1, 128) for b in (b1_0, b1_1, b1_2)]
  h = pl.pallas_call(
      _l1_body,
      grid=(NP // TROW,),
      in_specs=([_row_spec(1)] * 3 + [_full_spec(1, 128)] * 3
                + [_full_spec(128, 128)] * 3 + [_row_spec(W)] * 24),
      out_specs=_row_spec(128),
      out_shape=jax.ShapeDtypeStruct((NP, 128), jnp.float32),
  )(idg[0], idg[1], idg[2], *b1, W1_0, W1_1, W1_2, *aggs1)

  qs = pl.pallas_call(
      _l2pre_body,
      grid=(NP // TROW,),
      in_specs=([_row_spec(1)] * 3 + [_full_spec(128, 40)]
                + [_full_spec(128, 128)] * 3 + [_row_spec(128)]),
      out_specs=[_row_spec(W)] * 9,
      out_shape=[jax.ShapeDtypeStruct((NP, W), jnp.float32)] * 9,
  )(od[0], od[1], od[2], Wc, W2_0, W2_1, W2_2, h)

  aggs2 = (list(_agg_l2a(zer2d, *qs, s0, s1, s2, d0, d1, d2))
           + list(_agg_l2b(zer2d, *qs, s0, s1, s2, d0, d1, d2)))

  b2 = [b.reshape(1, 128) for b in (b2_0, b2_1, b2_2)]
  logits = pl.pallas_call(
      _final_body,
      grid=(NP // TROW,),
      in_specs=([_row_spec(1)] * 3 + [_full_spec(1, 128)] * 3
                + [_full_spec(128, 40)] + [_full_spec(1, 40)]
                + [_row_spec(W)] * 9),
      out_specs=_row_spec(40),
      out_shape=jax.ShapeDtypeStruct((NP, 40), jnp.float32),
  )(idg[0], idg[1], idg[2], *b2, Wc, bc.reshape(1, 40), *aggs2)

  return logits[:N]
